# baseline (speedup 1.0000x reference)
import functools

import jax
import jax.numpy as jnp
from jax.experimental import pallas as pl

_L = 2048
_C = 768
_K = 7
_LB = 128


def _silu(v):
    return v * jax.nn.sigmoid(v)


def _block_kernel(xpad_ref, vwin_ref, dwT_ref, pw_ref, woff_ref, wmask_ref,
                  boffT_ref, bmsT_ref, env_ref, kwtT_ref, wout_ref, boutr_ref,
                  out_ref, reg_ref, ent_ref):
    i = pl.program_id(0)
    l0 = i * _LB
    f32 = jnp.float32

    slab = xpad_ref[pl.ds(l0, _LB + 8), :]
    h = (slab[0:_LB] * dwT_ref[0:1, :] + slab[1:_LB + 1] * dwT_ref[1:2, :]
         + slab[2:_LB + 2] * dwT_ref[2:3, :] + dwT_ref[3:4, :])
    h = _silu(h)
    xdwT = jax.lax.dot_general(pw_ref[...], h, (((1,), (1,)), ((), ())),
                               preferred_element_type=f32) \
        + boffT_ref[:, 7:8]

    lrow = (jax.lax.broadcasted_iota(jnp.int32, (1, _LB), 1)
            + l0).astype(f32)

    sqacc = jnp.zeros((_C, _LB), f32)
    num = jnp.zeros((_C, _LB), f32)
    t1 = jnp.zeros((_C, _LB), f32)
    ssum = jnp.zeros((_C, _LB), f32)
    for k in range(_K):
        wk = woff_ref[:, k * _C:(k + 1) * _C]
        mo = jax.lax.dot_general(wk, xdwT, (((1,), (0,)), ((), ())),
                                 preferred_element_type=f32)
        off = jnp.tanh(mo + boffT_ref[:, k:k + 1]) * 2.0
        sqacc = sqacc + off * off
        j0 = 5 if k == _K - 1 else k
        nseg = 5 if k == _K - 1 else 4
        ubase = jnp.float32(k - 3 - (j0 - 5))
        lo = -(lrow + jnp.float32(j0 - 5))
        hi = jnp.float32(_L - 1) + lo
        u = jnp.minimum(jnp.maximum(off + ubase, lo), hi)
        vbase = jnp.broadcast_to(vwin_ref[j0:j0 + 1, :], (_C, _LB))
        acc = vbase + jnp.broadcast_to(
            vwin_ref[16 + j0:17 + j0, :], (_C, _LB)) * jnp.minimum(u, 1.0)
        for j in range(1, nseg):
            c = jnp.maximum(u - jnp.float32(j), 0.0)
            if j < nseg - 1:
                c = jnp.minimum(c, 1.0)
            acc = acc + jnp.broadcast_to(
                vwin_ref[16 + j0 + j:17 + j0 + j, :], (_C, _LB)) * c
        wmk = wmask_ref[:, k * _C:(k + 1) * _C]
        mmk = jax.lax.dot_general(wmk, xdwT, (((1,), (0,)), ((), ())),
                                  preferred_element_type=f32)
        envk = jnp.broadcast_to(env_ref[k:k + 1, 0:1], (_C, _LB))
        mk = mmk * envk + bmsT_ref[:, k:k + 1]
        ek = jnp.exp(mk)
        ssum = ssum + ek
        t1 = t1 + ek * mk
        num = num + (ek * acc) * kwtT_ref[:, k:k + 1]

    reg_blk = jnp.sum(sqacc)
    inv_s = 1.0 / ssum
    ent_blk = jnp.sum(t1 * inv_s - jnp.log(ssum))

    out_preT = num * inv_s
    res = jax.lax.dot_general(out_preT, wout_ref[...],
                              (((0,), (1,)), ((), ())),
                              preferred_element_type=f32)
    out_ref[...] = res + boutr_ref[0:1, :]

    reg_blk = (reg_blk / jnp.float32(_L * _C * _K)).reshape(1, 1)
    ent_blk = (ent_blk / jnp.float32(_L * _C)).reshape(1, 1)

    @pl.when(i == 0)
    def _():
        reg_ref[...] = reg_blk
        ent_ref[...] = ent_blk

    @pl.when(i != 0)
    def _():
        reg_ref[...] = reg_ref[...] + reg_blk
        ent_ref[...] = ent_ref[...] + ent_blk


@functools.partial(jax.jit, static_argnames=("interpret",))
def _run(x, params, interpret=False):
    p = params
    f32 = jnp.float32

    sigma = jnp.clip(jax.nn.softplus(p["raw_sigma"]), 0.05, 0.5)
    grid = jnp.linspace(-0.5, 0.5, _K).reshape(_K, 1)
    dist_sq = (grid / jnp.clip(sigma.reshape(1, 1), 1e-6, None)) ** 2
    env = jnp.exp(-0.5 * dist_sq.sum(-1))
    env = env / jnp.clip(env.sum(), 1e-8, None)
    env8 = jnp.zeros((8, 128), f32).at[:_K, :].set(env[:, None])

    kh = grid * 30.0
    kh = _silu(kh @ p["kn_W1"].T + p["kn_b1"])
    kh = _silu(kh @ p["kn_W2"].T + p["kn_b2"])
    kh = _silu(kh @ p["kn_W3"].T + p["kn_b3"])
    kernel_weights = kh @ p["kn_W4"].T + p["kn_b4"]
    kwtT = jnp.zeros((_C, 8), f32).at[:, :_K].set(
        kernel_weights.reshape(_C, _K))

    woff2 = p["W_off"].reshape(_C, _K * _C)
    wmask2 = p["W_mask"].reshape(_C, _K * _C)
    boffT = jnp.zeros((_C, 8), f32).at[:, :_K].set(p["b_off"].reshape(_C, _K))
    boffT = boffT.at[:, 7].set(p["pw_b"])
    bmsT = jnp.zeros((_C, 8), f32).at[:, :_K].set(
        p["b_mask"].reshape(_C, _K) * env[None, :])

    dwT = jnp.concatenate(
        [p["dw_w"][:, 0, :].T, p["dw_b"].reshape(1, _C),
         jnp.zeros((4, _C), f32)], axis=0)
    pw = p["pw_w"][:, :, 0]
    wout = p["W_out"]
    boutr = p["b_out"].reshape(1, _C)

    v = (x[0, 0:3, :] @ p["W_in"].T + p["b_in"]).reshape(-1)[:_L]
    vp = jnp.pad(v, (5, 6), mode="edge")
    vw = jnp.stack([vp[j:j + _L] for j in range(12)], axis=0)
    vwinT = jnp.zeros((32, _L), f32).at[:12, :].set(vw) \
        .at[16:27, :].set(vw[1:] - vw[:-1])

    xpad = jnp.concatenate(
        [jnp.zeros((1, _C), f32), x[0], jnp.zeros((7, _C), f32)], axis=0)

    nblk = _L // _LB
    full = lambda shape: pl.BlockSpec(shape, lambda i: (0, 0))
    out, reg, ent = pl.pallas_call(
        _block_kernel,
        grid=(nblk,),
        in_specs=[
            full((_L + 8, _C)),
            pl.BlockSpec((32, _LB), lambda i: (0, i)),
            full((8, _C)),
            full((_C, _C)),
            full((_C, _K * _C)),
            full((_C, _K * _C)),
            full((_C, 8)),
            full((_C, 8)),
            full((8, 128)),
            full((_C, 8)),
            full((_C, _C)),
            full((1, _C)),
        ],
        out_specs=[
            pl.BlockSpec((_LB, _C), lambda i: (i, 0)),
            pl.BlockSpec((1, 1), lambda i: (0, 0)),
            pl.BlockSpec((1, 1), lambda i: (0, 0)),
        ],
        out_shape=[
            jax.ShapeDtypeStruct((_L, _C), f32),
            jax.ShapeDtypeStruct((1, 1), f32),
            jax.ShapeDtypeStruct((1, 1), f32),
        ],
        interpret=interpret,
    )(xpad, vwinT, dwT, pw, woff2, wmask2, boffT, bmsT, env8, kwtT,
      wout, boutr)

    return out.reshape(1, _L, _C), reg[0, 0], ent[0, 0]


def kernel(x, params):
    return _run(x, params)


# speedup vs baseline: 1.1843x; 1.1843x over previous
import functools

import jax
import jax.numpy as jnp
from jax.experimental import pallas as pl

_L = 2048
_C = 768
_K = 7
_LB1 = 256
_LB = 256


def _silu(v):
    return v * jax.nn.sigmoid(v)


def _xdw_kernel(xpad_ref, dwT_ref, pw_ref, pwb_ref, xdwT_ref):
    i = pl.program_id(0)
    l0 = i * _LB1
    slab = xpad_ref[pl.ds(l0, _LB1 + 8), :]
    h = (slab[0:_LB1] * dwT_ref[0:1, :] + slab[1:_LB1 + 1] * dwT_ref[1:2, :]
         + slab[2:_LB1 + 2] * dwT_ref[2:3, :] + dwT_ref[3:4, :])
    h = _silu(h)
    xdwT_ref[...] = jax.lax.dot_general(
        pw_ref[...], h, (((1,), (1,)), ((), ())),
        preferred_element_type=jnp.float32) + pwb_ref[:, 0:1]


def _main_kernel(xdwT_ref, vwin_ref, woff_ref, wmask_ref,
                 boffT_ref, bmsT_ref, env_ref, kwtT_ref, wout_ref, boutr_ref,
                 out_ref, reg_ref, ent_ref):
    i = pl.program_id(0)
    l0 = i * _LB
    f32 = jnp.float32

    xdwT = xdwT_ref[...]
    lrow = (jax.lax.broadcasted_iota(jnp.int32, (1, _LB), 1)
            + l0).astype(f32)

    sqacc = jnp.zeros((_C, _LB), f32)
    num = jnp.zeros((_C, _LB), f32)
    t1 = jnp.zeros((_C, _LB), f32)
    ssum = jnp.zeros((_C, _LB), f32)
    for k in range(_K):
        wk = woff_ref[:, k * _C:(k + 1) * _C]
        mo = jax.lax.dot_general(wk, xdwT, (((1,), (0,)), ((), ())),
                                 preferred_element_type=f32)
        off = jnp.tanh(mo + boffT_ref[:, k:k + 1]) * 2.0
        sqacc = sqacc + off * off
        j0 = 5 if k == _K - 1 else k
        nseg = 5 if k == _K - 1 else 4
        ubase = jnp.float32(k - 3 - (j0 - 5))
        lo = -(lrow + jnp.float32(j0 - 5))
        hi = jnp.float32(_L - 1) + lo
        u = jnp.minimum(jnp.maximum(off + ubase, lo), hi)
        vbase = jnp.broadcast_to(vwin_ref[j0:j0 + 1, :], (_C, _LB))
        acc = vbase + jnp.broadcast_to(
            vwin_ref[16 + j0:17 + j0, :], (_C, _LB)) * jnp.minimum(u, 1.0)
        for j in range(1, nseg):
            c = jnp.maximum(u - jnp.float32(j), 0.0)
            if j < nseg - 1:
                c = jnp.minimum(c, 1.0)
            acc = acc + jnp.broadcast_to(
                vwin_ref[16 + j0 + j:17 + j0 + j, :], (_C, _LB)) * c
        wmk = wmask_ref[:, k * _C:(k + 1) * _C]
        mmk = jax.lax.dot_general(wmk, xdwT, (((1,), (0,)), ((), ())),
                                  preferred_element_type=f32)
        envk = jnp.broadcast_to(env_ref[k:k + 1, 0:1], (_C, _LB))
        mk = mmk * envk + bmsT_ref[:, k:k + 1]
        ek = jnp.exp(mk)
        ssum = ssum + ek
        t1 = t1 + ek * mk
        num = num + (ek * acc) * kwtT_ref[:, k:k + 1]

    reg_blk = jnp.sum(sqacc)
    inv_s = 1.0 / ssum
    ent_blk = jnp.sum(t1 * inv_s - jnp.log(ssum))

    out_preT = num * inv_s
    res = jax.lax.dot_general(out_preT, wout_ref[...],
                              (((0,), (1,)), ((), ())),
                              preferred_element_type=f32)
    out_ref[...] = res + boutr_ref[0:1, :]

    reg_blk = (reg_blk / jnp.float32(_L * _C * _K)).reshape(1, 1)
    ent_blk = (ent_blk / jnp.float32(_L * _C)).reshape(1, 1)

    @pl.when(i == 0)
    def _():
        reg_ref[...] = reg_blk
        ent_ref[...] = ent_blk

    @pl.when(i != 0)
    def _():
        reg_ref[...] = reg_ref[...] + reg_blk
        ent_ref[...] = ent_ref[...] + ent_blk


@functools.partial(jax.jit, static_argnames=("interpret",))
def _run(x, params, interpret=False):
    p = params
    f32 = jnp.float32

    sigma = jnp.clip(jax.nn.softplus(p["raw_sigma"]), 0.05, 0.5)
    grid = jnp.linspace(-0.5, 0.5, _K).reshape(_K, 1)
    dist_sq = (grid / jnp.clip(sigma.reshape(1, 1), 1e-6, None)) ** 2
    env = jnp.exp(-0.5 * dist_sq.sum(-1))
    env = env / jnp.clip(env.sum(), 1e-8, None)
    env8 = jnp.zeros((8, 128), f32).at[:_K, :].set(env[:, None])

    kh = grid * 30.0
    kh = _silu(kh @ p["kn_W1"].T + p["kn_b1"])
    kh = _silu(kh @ p["kn_W2"].T + p["kn_b2"])
    kh = _silu(kh @ p["kn_W3"].T + p["kn_b3"])
    kernel_weights = kh @ p["kn_W4"].T + p["kn_b4"]
    kwtT = jnp.zeros((_C, 8), f32).at[:, :_K].set(
        kernel_weights.reshape(_C, _K))

    woff2 = p["W_off"].reshape(_C, _K * _C)
    wmask2 = p["W_mask"].reshape(_C, _K * _C)
    boffT = jnp.zeros((_C, 8), f32).at[:, :_K].set(p["b_off"].reshape(_C, _K))
    bmsT = jnp.zeros((_C, 8), f32).at[:, :_K].set(
        p["b_mask"].reshape(_C, _K) * env[None, :])

    dwT = jnp.concatenate(
        [p["dw_w"][:, 0, :].T, p["dw_b"].reshape(1, _C),
         jnp.zeros((4, _C), f32)], axis=0)
    pw = p["pw_w"][:, :, 0]
    pwb = jnp.zeros((_C, 8), f32).at[:, 0].set(p["pw_b"])
    wout = p["W_out"]
    boutr = p["b_out"].reshape(1, _C)

    v = (x[0, 0:3, :] @ p["W_in"].T + p["b_in"]).reshape(-1)[:_L]
    vp = jnp.pad(v, (5, 6), mode="edge")
    vw = jnp.stack([vp[j:j + _L] for j in range(12)], axis=0)
    vwinT = jnp.zeros((32, _L), f32).at[:12, :].set(vw) \
        .at[16:27, :].set(vw[1:] - vw[:-1])

    xpad = jnp.concatenate(
        [jnp.zeros((1, _C), f32), x[0], jnp.zeros((7, _C), f32)], axis=0)

    full = lambda shape: pl.BlockSpec(shape, lambda i: (0, 0))

    xdwT = pl.pallas_call(
        _xdw_kernel,
        grid=(_L // _LB1,),
        in_specs=[
            full((_L + 8, _C)),
            full((8, _C)),
            full((_C, _C)),
            full((_C, 8)),
        ],
        out_specs=pl.BlockSpec((_C, _LB1), lambda i: (0, i)),
        out_shape=jax.ShapeDtypeStruct((_C, _L), f32),
        interpret=interpret,
    )(xpad, dwT, pw, pwb)

    out, reg, ent = pl.pallas_call(
        _main_kernel,
        grid=(_L // _LB,),
        in_specs=[
            pl.BlockSpec((_C, _LB), lambda i: (0, i)),
            pl.BlockSpec((32, _LB), lambda i: (0, i)),
            full((_C, _K * _C)),
            full((_C, _K * _C)),
            full((_C, 8)),
            full((_C, 8)),
            full((8, 128)),
            full((_C, 8)),
            full((_C, _C)),
            full((1, _C)),
        ],
        out_specs=[
            pl.BlockSpec((_LB, _C), lambda i: (i, 0)),
            pl.BlockSpec((1, 1), lambda i: (0, 0)),
            pl.BlockSpec((1, 1), lambda i: (0, 0)),
        ],
        out_shape=[
            jax.ShapeDtypeStruct((_L, _C), f32),
            jax.ShapeDtypeStruct((1, 1), f32),
            jax.ShapeDtypeStruct((1, 1), f32),
        ],
        interpret=interpret,
    )(xdwT, vwinT, woff2, wmask2, boffT, bmsT, env8, kwtT, wout, boutr)

    return out.reshape(1, _L, _C), reg[0, 0], ent[0, 0]


def kernel(x, params):
    return _run(x, params)


# speedup vs baseline: 1.1927x; 1.0071x over previous
import functools

import jax
import jax.numpy as jnp
from jax.experimental import pallas as pl

_L = 2048
_C = 768
_K = 7
_LB1 = 256
_LB = 256


def _silu(v):
    return v * jax.nn.sigmoid(v)


def _xdw_kernel(xpad_ref, dwT_ref, pw_ref, pwb_ref, xdwT_ref):
    i = pl.program_id(0)
    l0 = i * _LB1
    slab = xpad_ref[pl.ds(l0, _LB1 + 8), :]
    h = (slab[0:_LB1] * dwT_ref[0:1, :] + slab[1:_LB1 + 1] * dwT_ref[1:2, :]
         + slab[2:_LB1 + 2] * dwT_ref[2:3, :] + dwT_ref[3:4, :])
    h = _silu(h)
    xdwT_ref[...] = jax.lax.dot_general(
        pw_ref[...], h, (((1,), (1,)), ((), ())),
        preferred_element_type=jnp.float32) + pwb_ref[:, 0:1]


def _main_kernel(xdwT_ref, vwin_ref, woff_ref, wmask_ref,
                 boffT_ref, bmsT_ref, env_ref, kwtT_ref, wout_ref, boutr_ref,
                 out_ref, reg_ref, ent_ref):
    i = pl.program_id(0)
    l0 = i * _LB
    f32 = jnp.float32

    xdwT = xdwT_ref[...]
    lrow = (jax.lax.broadcasted_iota(jnp.int32, (1, _LB), 1)
            + l0).astype(f32)

    sqacc = jnp.zeros((_C, _LB), f32)
    num = jnp.zeros((_C, _LB), f32)
    t1 = jnp.zeros((_C, _LB), f32)
    ssum = jnp.zeros((_C, _LB), f32)
    for k in range(_K):
        wk = woff_ref[:, k * _C:(k + 1) * _C]
        mo = jax.lax.dot_general(wk, xdwT, (((1,), (0,)), ((), ())),
                                 preferred_element_type=f32)
        off = jnp.tanh(mo + boffT_ref[:, k:k + 1]) * 2.0
        sqacc = sqacc + off * off
        j0 = 5 if k == _K - 1 else k
        nseg = 5 if k == _K - 1 else 4
        ubase = jnp.float32(k - 3 - (j0 - 5))
        lo = -(lrow + jnp.float32(j0 - 5))
        hi = jnp.float32(_L - 1) + lo
        u = jnp.minimum(jnp.maximum(off + ubase, lo), hi)
        acc = jnp.broadcast_to(vwin_ref[j0:j0 + 1, :], (_C, _LB))
        for j in range(1, nseg + 1):
            row = (12 + j0 + nseg - 1) if j == nseg else (23 + j0 + j - 1)
            acc = acc + jnp.broadcast_to(
                vwin_ref[row:row + 1, :], (_C, _LB)) \
                * jnp.minimum(u, jnp.float32(j))
        wmk = wmask_ref[:, k * _C:(k + 1) * _C]
        mmk = jax.lax.dot_general(wmk, xdwT, (((1,), (0,)), ((), ())),
                                  preferred_element_type=f32)
        envk = jnp.broadcast_to(env_ref[k:k + 1, 0:1], (_C, _LB))
        mk = mmk * envk + bmsT_ref[:, k:k + 1]
        ek = jnp.exp(mk)
        ssum = ssum + ek
        t1 = t1 + ek * mk
        num = num + (ek * acc) * kwtT_ref[:, k:k + 1]

    reg_blk = jnp.sum(sqacc)
    inv_s = 1.0 / ssum
    ent_blk = jnp.sum(t1 * inv_s - jnp.log(ssum))

    out_preT = num * inv_s
    res = jax.lax.dot_general(out_preT, wout_ref[...],
                              (((0,), (1,)), ((), ())),
                              preferred_element_type=f32)
    out_ref[...] = res + boutr_ref[0:1, :]

    reg_blk = (reg_blk / jnp.float32(_L * _C * _K)).reshape(1, 1)
    ent_blk = (ent_blk / jnp.float32(_L * _C)).reshape(1, 1)

    @pl.when(i == 0)
    def _():
        reg_ref[...] = reg_blk
        ent_ref[...] = ent_blk

    @pl.when(i != 0)
    def _():
        reg_ref[...] = reg_ref[...] + reg_blk
        ent_ref[...] = ent_ref[...] + ent_blk


@functools.partial(jax.jit, static_argnames=("interpret",))
def _run(x, params, interpret=False):
    p = params
    f32 = jnp.float32

    sigma = jnp.clip(jax.nn.softplus(p["raw_sigma"]), 0.05, 0.5)
    grid = jnp.linspace(-0.5, 0.5, _K).reshape(_K, 1)
    dist_sq = (grid / jnp.clip(sigma.reshape(1, 1), 1e-6, None)) ** 2
    env = jnp.exp(-0.5 * dist_sq.sum(-1))
    env = env / jnp.clip(env.sum(), 1e-8, None)
    env8 = jnp.zeros((8, 128), f32).at[:_K, :].set(env[:, None])

    kh = grid * 30.0
    kh = _silu(kh @ p["kn_W1"].T + p["kn_b1"])
    kh = _silu(kh @ p["kn_W2"].T + p["kn_b2"])
    kh = _silu(kh @ p["kn_W3"].T + p["kn_b3"])
    kernel_weights = kh @ p["kn_W4"].T + p["kn_b4"]
    kwtT = jnp.zeros((_C, 8), f32).at[:, :_K].set(
        kernel_weights.reshape(_C, _K))

    woff2 = p["W_off"].reshape(_C, _K * _C)
    wmask2 = p["W_mask"].reshape(_C, _K * _C)
    boffT = jnp.zeros((_C, 8), f32).at[:, :_K].set(p["b_off"].reshape(_C, _K))
    bmsT = jnp.zeros((_C, 8), f32).at[:, :_K].set(
        p["b_mask"].reshape(_C, _K) * env[None, :])

    dwT = jnp.concatenate(
        [p["dw_w"][:, 0, :].T, p["dw_b"].reshape(1, _C),
         jnp.zeros((4, _C), f32)], axis=0)
    pw = p["pw_w"][:, :, 0]
    pwb = jnp.zeros((_C, 8), f32).at[:, 0].set(p["pw_b"])
    wout = p["W_out"]
    boutr = p["b_out"].reshape(1, _C)

    v = (x[0, 0:3, :] @ p["W_in"].T + p["b_in"]).reshape(-1)[:_L]
    vp = jnp.pad(v, (5, 6), mode="edge")
    vw = jnp.stack([vp[j:j + _L] for j in range(12)], axis=0)
    dv = vw[1:] - vw[:-1]
    vwinT = jnp.concatenate([vw, dv, dv[:-2] - dv[1:-1]], axis=0)

    xpad = jnp.concatenate(
        [jnp.zeros((1, _C), f32), x[0], jnp.zeros((7, _C), f32)], axis=0)

    full = lambda shape: pl.BlockSpec(shape, lambda i: (0, 0))

    xdwT = pl.pallas_call(
        _xdw_kernel,
        grid=(_L // _LB1,),
        in_specs=[
            full((_L + 8, _C)),
            full((8, _C)),
            full((_C, _C)),
            full((_C, 8)),
        ],
        out_specs=pl.BlockSpec((_C, _LB1), lambda i: (0, i)),
        out_shape=jax.ShapeDtypeStruct((_C, _L), f32),
        interpret=interpret,
    )(xpad, dwT, pw, pwb)

    out, reg, ent = pl.pallas_call(
        _main_kernel,
        grid=(_L // _LB,),
        in_specs=[
            pl.BlockSpec((_C, _LB), lambda i: (0, i)),
            pl.BlockSpec((32, _LB), lambda i: (0, i)),
            full((_C, _K * _C)),
            full((_C, _K * _C)),
            full((_C, 8)),
            full((_C, 8)),
            full((8, 128)),
            full((_C, 8)),
            full((_C, _C)),
            full((1, _C)),
        ],
        out_specs=[
            pl.BlockSpec((_LB, _C), lambda i: (i, 0)),
            pl.BlockSpec((1, 1), lambda i: (0, 0)),
            pl.BlockSpec((1, 1), lambda i: (0, 0)),
        ],
        out_shape=[
            jax.ShapeDtypeStruct((_L, _C), f32),
            jax.ShapeDtypeStruct((1, 1), f32),
            jax.ShapeDtypeStruct((1, 1), f32),
        ],
        interpret=interpret,
    )(xdwT, vwinT, woff2, wmask2, boffT, bmsT, env8, kwtT, wout, boutr)

    return out.reshape(1, _L, _C), reg[0, 0], ent[0, 0]


def kernel(x, params):
    return _run(x, params)
